# baseline (speedup 1.0000x reference)
import functools

import jax
import jax.numpy as jnp
from jax.experimental import pallas as pl
from jax.experimental.pallas import tpu as pltpu

PMASK_SLOPE = 5.0
SAMPLE_SLOPE = 12.0
SPARSITY = 0.25


def _body(example_ref, weight_ref, thresh_ref, out_ref):
    p = jax.nn.sigmoid(weight_ref[...] * PMASK_SLOPE)
    pbar = jnp.mean(p)
    pmask = jnp.where(
        pbar > SPARSITY,
        p * (SPARSITY / pbar),
        1.0 - (1.0 - p) * ((1.0 - SPARSITY) / (1.0 - pbar)),
    )
    mask = jax.nn.sigmoid((pmask - thresh_ref[0]) * SAMPLE_SLOPE)
    out_ref[...] = example_ref[...] * mask[None, :, :]


@functools.partial(jax.jit, static_argnames=("row_block",))
def _loupe_mul(example3, weight2, thresh3, row_block):
    B, H, W = example3.shape
    grid = (B, H // row_block)
    return pl.pallas_call(
        _body,
        grid=grid,
        in_specs=[
            pl.BlockSpec((1, row_block, W), lambda b, r: (b, r, 0)),
            pl.BlockSpec((1, W), lambda b, r: (0, 0)),
            pl.BlockSpec((1, 1, W), lambda b, r: (b, 0, 0)),
        ],
        out_specs=pl.BlockSpec((1, row_block, W), lambda b, r: (b, r, 0)),
        out_shape=jax.ShapeDtypeStruct((B, H, W), example3.dtype),
        compiler_params=pltpu.CompilerParams(
            dimension_semantics=("parallel", "parallel"),
        ),
    )(example3, weight2, thresh3)


def kernel(example, weight, thresh):
    B, C, H, W = example.shape
    out = _loupe_mul(example.reshape(B, H, W), weight.reshape(1, W),
                     thresh.reshape(B, 1, W), row_block=512)
    return out.reshape(B, C, H, W)
